# Initial kernel scaffold; baseline (speedup 1.0000x reference)
import jax
import jax.numpy as jnp
from jax.experimental import pallas as pl


def kernel(x, edge_index, batch, l0_eps, l0_W1, l0_b1, l0_W2, l0_b2, l0_gamma, l0_beta, l1_eps, l1_W1, l1_b1, l1_W2, l1_b2, l1_gamma, l1_beta, l2_eps, l2_W1, l2_b1, l2_W2, l2_b2, l2_gamma, l2_beta, head_W1, head_b1, head_W2, head_b2):
    raise NotImplementedError("write your pallas kernel here")


# speedup vs baseline: 5.3490x; 5.3490x over previous
import functools

import jax
import jax.numpy as jnp
from jax import lax
from jax.experimental import pallas as pl
from jax.experimental.pallas import tpu as pltpu
from jax.experimental.pallas import tpu_sc as plsc

NC = 2
NS = 16
NW = NC * NS


def _sc_segment_sum(values, dst, num_segments, src=None, chunk=128):
    n_rows, d = values.shape
    n_edges = dst.shape[0]
    n_chunks = n_edges // chunk
    rpt = (num_segments // NS) // 8 * 8
    tail = num_segments - rpt * NS
    dst2 = dst
    src2 = src
    zeros = jnp.zeros((num_segments, d), jnp.float32)
    mesh = plsc.VectorSubcoreMesh(core_axis_name="c", subcore_axis_name="s")

    scratch = [
        pltpu.VMEM((1, chunk), jnp.int32),
        pltpu.VMEM((chunk, d), jnp.float32),
        pltpu.VMEM_SHARED((num_segments, d), jnp.float32),
        pltpu.SemaphoreType.DMA,
    ]
    if src2 is not None:
        scratch.append(pltpu.VMEM((1, chunk), jnp.int32))

    def body(val_hbm, dst_hbm, zero_hbm, *rest):
        if src2 is not None:
            src_hbm, out_hbm, dst_v, rows_v, acc, sem, src_v = rest
        else:
            out_hbm, dst_v, rows_v, acc, sem = rest
        cid = lax.axis_index("c")
        sid = lax.axis_index("s")
        wid = sid * NC + cid

        pltpu.sync_copy(zero_hbm.at[pl.ds(sid * rpt, rpt)],
                        acc.at[pl.ds(sid * rpt, rpt)])
        if tail:
            @pl.when(sid == 0)
            def _():
                pltpu.sync_copy(zero_hbm.at[pl.ds(rpt * NS, tail)],
                                acc.at[pl.ds(rpt * NS, tail)])
        plsc.subcore_barrier()

        n_my = (n_chunks - wid + NW - 1) // NW

        def step(i, carry):
            c = wid + i * NW
            off = c * chunk
            pltpu.sync_copy(dst_hbm.at[pl.ds(off, chunk)], dst_v.at[0])
            if src2 is not None:
                pltpu.sync_copy(src_hbm.at[pl.ds(off, chunk)], src_v.at[0])
                pltpu.async_copy(val_hbm.at[src_v.at[0]], rows_v, sem).wait()
            else:
                pltpu.sync_copy(val_hbm.at[pl.ds(off, chunk)], rows_v)
            pltpu.sync_copy(rows_v, acc.at[dst_v.at[0]], add=True)
            return carry

        lax.fori_loop(0, n_my, step, 0)
        plsc.subcore_barrier()
        pltpu.sync_copy(acc.at[pl.ds(sid * rpt, rpt)],
                        out_hbm.at[cid, pl.ds(sid * rpt, rpt)])
        if tail:
            @pl.when(sid == 0)
            def _():
                pltpu.sync_copy(acc.at[pl.ds(rpt * NS, tail)],
                                out_hbm.at[cid, pl.ds(rpt * NS, tail)])

    args = (values, dst2, zeros) + (() if src2 is None else (src2,))
    return pl.kernel(
        body,
        out_type=jax.ShapeDtypeStruct((NC, num_segments, d), jnp.float32),
        mesh=mesh,
        scratch_types=scratch,
    )(*args)


def _tc_layer(h, parts, eps, w1, b1, w2, b2, gamma, beta):
    n, d = h.shape

    def body(eps_ref, h_ref, p_ref, w1_ref, b1_ref, w2_ref, b2_ref,
             g_ref, be_ref, o_ref):
        hv = h_ref[...]
        z = hv + eps_ref[0] * hv + p_ref[0] + p_ref[1]
        a = jnp.dot(z, w1_ref[...], preferred_element_type=jnp.float32,
                    precision=lax.Precision.HIGHEST) + b1_ref[...]
        a = jnp.maximum(a, 0.0)
        z2 = jnp.dot(a, w2_ref[...], preferred_element_type=jnp.float32,
                     precision=lax.Precision.HIGHEST) + b2_ref[...]
        mu = jnp.mean(z2, axis=0, keepdims=True)
        var = jnp.mean(z2 * z2, axis=0, keepdims=True) - mu * mu
        zn = (z2 - mu) * lax.rsqrt(var + 1e-5) * g_ref[...] + be_ref[...]
        o_ref[...] = jnp.maximum(zn, 0.0) + hv

    smem = pl.BlockSpec(memory_space=pltpu.SMEM)
    vmem = pl.BlockSpec(memory_space=pltpu.VMEM)
    return pl.pallas_call(
        body,
        out_shape=jax.ShapeDtypeStruct((n, d), jnp.float32),
        in_specs=[smem] + [vmem] * 8,
        out_specs=vmem,
    )(jnp.reshape(eps, (1,)), h, parts, w1, jnp.reshape(b1, (1, d)), w2,
      jnp.reshape(b2, (1, d)), jnp.reshape(gamma, (1, d)),
      jnp.reshape(beta, (1, d)))


def _tc_head(parts, w1, b1, w2, b2):
    g, d = parts.shape[1], parts.shape[2]
    d_out = w2.shape[1]

    def body(p_ref, w1_ref, b1_ref, w2_ref, b2_ref, o_ref):
        zp = p_ref[0] + p_ref[1]
        a = jnp.dot(zp, w1_ref[...], preferred_element_type=jnp.float32,
                    precision=lax.Precision.HIGHEST) + b1_ref[...]
        a = jnp.maximum(a, 0.0)
        o_ref[...] = jnp.dot(a, w2_ref[...], preferred_element_type=jnp.float32,
                             precision=lax.Precision.HIGHEST) + b2_ref[...]

    vmem = pl.BlockSpec(memory_space=pltpu.VMEM)
    return pl.pallas_call(
        body,
        out_shape=jax.ShapeDtypeStruct((g, d_out), jnp.float32),
        in_specs=[vmem] * 5,
        out_specs=vmem,
    )(parts, w1, jnp.reshape(b1, (1, d)), w2, jnp.reshape(b2, (1, d_out)))


def kernel(x, edge_index, batch, l0_eps, l0_W1, l0_b1, l0_W2, l0_b2, l0_gamma,
           l0_beta, l1_eps, l1_W1, l1_b1, l1_W2, l1_b2, l1_gamma, l1_beta,
           l2_eps, l2_W1, l2_b1, l2_W2, l2_b2, l2_gamma, l2_beta, head_W1,
           head_b1, head_W2, head_b2):
    src = edge_index[0]
    dst = edge_index[1]
    n = x.shape[0]
    g = 512

    layers = [
        (l0_eps, l0_W1, l0_b1, l0_W2, l0_b2, l0_gamma, l0_beta),
        (l1_eps, l1_W1, l1_b1, l1_W2, l1_b2, l1_gamma, l1_beta),
        (l2_eps, l2_W1, l2_b1, l2_W2, l2_b2, l2_gamma, l2_beta),
    ]
    h = x
    for (eps, w1, b1, w2, b2, gamma, beta) in layers:
        parts = _sc_segment_sum(h, dst, n, src=src, chunk=128)
        h = _tc_layer(h, parts, eps, w1, b1, w2, b2, gamma, beta)

    pool_parts = _sc_segment_sum(h, batch, g, src=None, chunk=80)
    return _tc_head(pool_parts, head_W1, head_b1, head_W2, head_b2)
